# baseline (speedup 1.0000x reference)
import functools

import jax
import jax.numpy as jnp
from jax import lax
from jax.experimental import pallas as pl
from jax.experimental.pallas import tpu as pltpu
from jax.experimental.pallas import tpu_sc as plsc

B = 256
J = 17
S = 96 * 72
LN = J * S
TOPK = 8
BB = 8

NC, NS = 2, 16
NW = NC * NS
BSC = 256
BTC = B - BSC
SPW = BSC // NW
RING = 4
VL = 16


NCHUNK = SPW * J
UNR = 8


def _sc_stream_body(x_hbm, y_hbm, out_hbm, xbuf, ybuf, l16, xsem, ysem):
    wid = lax.axis_index("s") * NC + lax.axis_index("c")
    base = wid * SPW

    def start(c, slot):
        b = c // J
        j = c - b * J
        x_src = x_hbm.at[BTC + base + b, pl.ds(j * S, S)]
        y_src = y_hbm.at[BTC + base + b, pl.ds(j * S, S)]
        pltpu.make_async_copy(x_src, xbuf.at[slot], xsem.at[slot]).start()
        pltpu.make_async_copy(y_src, ybuf.at[slot], ysem.at[slot]).start()

    for s in range(RING):
        start(s, s)

    def group(it, carry):
        c0 = it * RING
        for u in range(RING):
            c = c0 + u
            b = c // J
            j = c - b * J
            pltpu.make_async_copy(
                x_hbm.at[BTC + base + b, pl.ds(j * S, S)],
                xbuf.at[u], xsem.at[u],
            ).wait()
            pltpu.make_async_copy(
                y_hbm.at[BTC + base + b, pl.ds(j * S, S)],
                ybuf.at[u], ysem.at[u],
            ).wait()

            def step(k, accs, slot=u):
                new = []
                for v in range(UNR):
                    xg = xbuf[slot, pl.ds((k * UNR + v) * VL, VL)]
                    yg = ybuf[slot, pl.ds((k * UNR + v) * VL, VL)]
                    d = xg - yg
                    new.append(accs[v] + d * d)
                return tuple(new)

            accs = lax.fori_loop(
                0, S // (VL * UNR), step,
                tuple(jnp.zeros((VL,), jnp.float32) for _ in range(UNR)),
            )
            acc = accs[0]
            for v in range(1, UNR):
                acc = acc + accs[v]
            l16[j, :] = acc

            @pl.when(c + RING < NCHUNK)
            def _():
                start(c + RING, u)

            @pl.when(j == J - 1)
            def _():
                pltpu.sync_copy(l16, out_hbm.at[:, base + b, :])

        return carry

    lax.fori_loop(0, NCHUNK // RING, group, 0)


_sc_stream = functools.partial(
    pl.kernel,
    out_type=jax.ShapeDtypeStruct((J, BSC, VL), jnp.float32),
    mesh=plsc.VectorSubcoreMesh(core_axis_name="c", subcore_axis_name="s"),
    scratch_types=[
        pltpu.VMEM((RING, S), jnp.float32),
        pltpu.VMEM((RING, S), jnp.float32),
        pltpu.VMEM((J, VL), jnp.float32),
        pltpu.SemaphoreType.DMA((RING,)),
        pltpu.SemaphoreType.DMA((RING,)),
    ],
)(_sc_stream_body)


def _tc_sums_body(x_ref, y_ref, o_ref):
    d = x_ref[...] - y_ref[...]
    d2 = d * d
    for j in range(J):
        s = jnp.sum(d2[:, j * S:(j + 1) * S], axis=1, keepdims=True)
        o_ref[:, j:j + 1] = s * (0.5 / S)


def _mine_body(sc_ref, o_ref):
    l = jnp.sum(sc_ref[...], axis=2) * (0.5 / S)
    jidx = jax.lax.broadcasted_iota(jnp.int32, (J, B), 0)
    rank = jnp.zeros((J, B), jnp.int32)
    for k in range(J):
        lk = l[k:k + 1, :]
        gt = (lk > l) | ((lk == l) & (k < jidx))
        rank = rank + gt.astype(jnp.int32)
    topsum = jnp.sum(jnp.where(rank < TOPK, l, 0.0))
    o_ref[...] = topsum[None, None] * (1.0 / (TOPK * B))


def kernel(output, target):
    x = output.reshape(B, LN)
    y = target.reshape(B, LN)
    sc16 = _sc_stream(x, y)
    out = pl.pallas_call(
        _mine_body,
        out_shape=jax.ShapeDtypeStruct((1, 1), jnp.float32),
    )(sc16)
    return out[0, 0]


# speedup vs baseline: 1.1559x; 1.1559x over previous
import functools

import jax
import jax.numpy as jnp
from jax import lax
from jax.experimental import pallas as pl
from jax.experimental.pallas import tpu as pltpu
from jax.experimental.pallas import tpu_sc as plsc

B = 256
J = 17
S = 96 * 72
LN = J * S
TOPK = 8
BB = 8

NC, NS = 2, 16
NW = NC * NS
BSC = 160
BTC = B - BSC
SPW = BSC // NW
RING = 4
VL = 16
UNR = 8


def _sc_stream_body(x_hbm, y_hbm, out_hbm, xbuf, ybuf, l16, xsem, ysem):
    wid = lax.axis_index("s") * NC + lax.axis_index("c")
    base = wid * SPW

    def start(c, slot):
        b = c // J
        j = c % J
        x_src = x_hbm.at[BTC + base + b, pl.ds(j * S, S)]
        y_src = y_hbm.at[BTC + base + b, pl.ds(j * S, S)]
        pltpu.make_async_copy(x_src, xbuf.at[slot], xsem.at[slot]).start()
        pltpu.make_async_copy(y_src, ybuf.at[slot], ysem.at[slot]).start()

    for s in range(RING):
        start(s, s)

    NCHUNK = SPW * J
    for c in range(NCHUNK):
        slot = c % RING
        b = c // J
        j = c % J
        x_src = x_hbm.at[BTC + base + b, pl.ds(j * S, S)]
        y_src = y_hbm.at[BTC + base + b, pl.ds(j * S, S)]
        pltpu.make_async_copy(x_src, xbuf.at[slot], xsem.at[slot]).wait()
        pltpu.make_async_copy(y_src, ybuf.at[slot], ysem.at[slot]).wait()

        def step(k, accs, slot=slot):
            new = []
            for u in range(UNR):
                xg = xbuf[slot, pl.ds((k * UNR + u) * VL, VL)]
                yg = ybuf[slot, pl.ds((k * UNR + u) * VL, VL)]
                d = xg - yg
                new.append(accs[u] + d * d)
            return tuple(new)

        accs = lax.fori_loop(
            0, S // (VL * UNR), step,
            tuple(jnp.zeros((VL,), jnp.float32) for _ in range(UNR)),
        )
        acc = accs[0]
        for u in range(1, UNR):
            acc = acc + accs[u]
        l16[j, :] = acc
        if c + RING < NCHUNK:
            start(c + RING, slot)
        if j == J - 1:
            pltpu.sync_copy(l16, out_hbm.at[:, base + b, :])


_sc_stream = functools.partial(
    pl.kernel,
    out_type=jax.ShapeDtypeStruct((J, BSC, VL), jnp.float32),
    mesh=plsc.VectorSubcoreMesh(core_axis_name="c", subcore_axis_name="s"),
    scratch_types=[
        pltpu.VMEM((RING, S), jnp.float32),
        pltpu.VMEM((RING, S), jnp.float32),
        pltpu.VMEM((J, VL), jnp.float32),
        pltpu.SemaphoreType.DMA((RING,)),
        pltpu.SemaphoreType.DMA((RING,)),
    ],
)(_sc_stream_body)


def _tc_sums_body(x_ref, y_ref, o_ref):
    d = x_ref[...] - y_ref[...]
    d2 = d * d
    for j in range(J):
        s = jnp.sum(d2[:, j * S:(j + 1) * S], axis=1, keepdims=True)
        o_ref[:, j:j + 1] = s * (0.5 / S)


def _mine_body(lt_ref, sc_ref, o_ref):
    l_sc = jnp.sum(sc_ref[...], axis=2) * (0.5 / S)
    l = jnp.concatenate([lt_ref[...], l_sc], axis=1)
    jidx = jax.lax.broadcasted_iota(jnp.int32, (J, B), 0)
    rank = jnp.zeros((J, B), jnp.int32)
    for k in range(J):
        lk = l[k:k + 1, :]
        gt = (lk > l) | ((lk == l) & (k < jidx))
        rank = rank + gt.astype(jnp.int32)
    topsum = jnp.sum(jnp.where(rank < TOPK, l, 0.0))
    o_ref[...] = topsum[None, None] * (1.0 / (TOPK * B))


def kernel(output, target):
    x = output.reshape(B, LN)
    y = target.reshape(B, LN)
    sc16 = _sc_stream(x, y)
    losses_tc = pl.pallas_call(
        _tc_sums_body,
        grid=(BTC // BB,),
        in_specs=[
            pl.BlockSpec((BB, LN), lambda i: (i, 0)),
            pl.BlockSpec((BB, LN), lambda i: (i, 0)),
        ],
        out_specs=pl.BlockSpec((BB, J), lambda i: (i, 0)),
        out_shape=jax.ShapeDtypeStruct((BTC, J), jnp.float32),
    )(x, y)
    out = pl.pallas_call(
        _mine_body,
        out_shape=jax.ShapeDtypeStruct((1, 1), jnp.float32),
    )(losses_tc.T, sc16)
    return out[0, 0]


# speedup vs baseline: 1.3519x; 1.1695x over previous
import jax
import jax.numpy as jnp
from jax.experimental import pallas as pl

B = 256
J = 17
S = 96 * 72
LN = J * S
TOPK = 8
BB = 8


def _sums_body(x_ref, y_ref, o_ref):
    d = x_ref[...] - y_ref[...]
    d2 = d * d
    for j in range(J):
        s = jnp.sum(d2[:, j * S:(j + 1) * S], axis=1, keepdims=True)
        o_ref[:, j:j + 1] = s * (0.5 / S)


def _mine_body(l_ref, o_ref):
    l = l_ref[...]
    jidx = jax.lax.broadcasted_iota(jnp.int32, (J, B), 0)
    rank = jnp.zeros((J, B), jnp.int32)
    for k in range(J):
        lk = l[k:k + 1, :]
        gt = (lk > l) | ((lk == l) & (k < jidx))
        rank = rank + gt.astype(jnp.int32)
    topsum = jnp.sum(jnp.where(rank < TOPK, l, 0.0))
    o_ref[...] = topsum[None, None] * (1.0 / (TOPK * B))


def kernel(output, target):
    x = output.reshape(B, LN)
    y = target.reshape(B, LN)
    losses = pl.pallas_call(
        _sums_body,
        grid=(B // BB,),
        in_specs=[
            pl.BlockSpec((BB, LN), lambda i: (i, 0)),
            pl.BlockSpec((BB, LN), lambda i: (i, 0)),
        ],
        out_specs=pl.BlockSpec((BB, J), lambda i: (i, 0)),
        out_shape=jax.ShapeDtypeStruct((B, J), jnp.float32),
    )(x, y)
    out = pl.pallas_call(
        _mine_body,
        out_shape=jax.ShapeDtypeStruct((1, 1), jnp.float32),
    )(losses.T)
    return out[0, 0]
